# baseline (speedup 1.0000x reference)
import functools
import math

import jax
import jax.numpy as jnp
import numpy as np
from jax import lax
from jax.experimental import pallas as pl
from jax.experimental.pallas import tpu as pltpu
from jax.experimental.pallas import tpu_sc as plsc

D_MODEL = 64
SEQ = 200
NT = 49
SCALE = 8.0
NC = 2
NS = 16
NW = NC * NS
CH = 128
PE_ROWS = SEQ + CH
LANES = 16


def _pe_scaled_ext():
    position = np.arange(0, SEQ, dtype=np.float32)[:, None]
    div = np.exp(
        np.arange(0, D_MODEL, 2, dtype=np.float32) * -(math.log(10000.0) / D_MODEL)
    )
    pe = np.zeros((SEQ, D_MODEL), dtype=np.float32)
    pe[:, 0::2] = np.sin(position * div)
    pe[:, 1::2] = np.cos(position * div)
    pe = pe * np.float32(SCALE)
    ext = np.concatenate([pe, pe, pe], axis=0)[:PE_ROWS]
    return jnp.asarray(ext)


def _make_sc_embed(n_rows):
    rows_per_w = n_rows // NW
    n_chunks = rows_per_w // CH
    n_pairs = n_chunks // 2
    mesh = plsc.VectorSubcoreMesh(core_axis_name="c", subcore_axis_name="s")

    @functools.partial(
        pl.kernel,
        out_type=jax.ShapeDtypeStruct((n_rows // 2, 2 * D_MODEL), jnp.float32),
        mesh=mesh,
        compiler_params=pltpu.CompilerParams(use_tc_tiling_on_sc=True),
        scratch_types=[
            pltpu.VMEM((2, CH), jnp.int32),
            pltpu.VMEM((2, CH + LANES), jnp.int32),
            pltpu.VMEM((2, CH, D_MODEL), jnp.float32),
            pltpu.VMEM((CH // 2, 2 * D_MODEL), jnp.float32),
            pltpu.VMEM((NT, D_MODEL), jnp.float32),
            pltpu.VMEM((PE_ROWS, D_MODEL), jnp.float32),
            pltpu.SemaphoreType.DMA,
            pltpu.SemaphoreType.DMA,
        ],
    )
    def sc_embed(xf, tf, tok_tab, tt8, pe8, out,
                 xi_v, t_v, tok_v, out_v, tt_v, pe_v, sem_t0, sem_t1):
        wid = lax.axis_index("s") * NC + lax.axis_index("c")
        base0 = wid * rows_per_w
        sems_t = (sem_t0, sem_t1)
        pltpu.sync_copy(tt8, tt_v)
        pltpu.sync_copy(pe8, pe_v)

        def load_and_issue(c, p):
            base = pl.multiple_of(base0 + c * CH, CH)
            pltpu.sync_copy(xf.at[pl.ds(base, CH)], xi_v.at[p])
            pltpu.sync_copy(tf.at[pl.ds(base, CH)], t_v.at[p, pl.ds(0, CH)])
            for k in range(CH // LANES):
                v = xi_v[p, pl.ds(k * LANES, LANES)]
                for u in range(LANES):
                    pltpu.async_copy(
                        tok_tab.at[v[u]], tok_v.at[p, k * LANES + u], sems_t[p]
                    )

        def drain(p):
            pltpu.make_async_copy(
                tok_tab.at[pl.ds(0, CH)], tok_v.at[p], sems_t[p]
            ).wait()

        def compute_store(c, p):
            base = pl.multiple_of(base0 + c * CH, CH)
            s_off = lax.rem(base, SEQ)

            def grp_body(g2, rcarry):
                rbase = pl.multiple_of(g2 * LANES, LANES)
                tv = t_v[p, pl.ds(rbase, LANES)]
                for u in range(LANES):
                    r = rbase + u
                    tr = tv[u]
                    pr = s_off + r
                    ohalf = (u & 1) * D_MODEL
                    orow = (rbase >> 1) + (u >> 1)
                    for j in range(D_MODEL // LANES):
                        sl = pl.ds(j * LANES, LANES)
                        out_v[orow, pl.ds(ohalf + j * LANES, LANES)] = (
                            tok_v[p, r, sl] * SCALE + tt_v[tr, sl] + pe_v[pr, sl]
                        )
                return rcarry

            lax.fori_loop(0, CH // LANES, grp_body, 0)
            pltpu.sync_copy(
                out_v, out.at[pl.ds(pl.multiple_of(base // 2, CH // 2), CH // 2)]
            )

        load_and_issue(0, 0)

        def pair_body(g, carry):
            load_and_issue(2 * g + 1, 1)
            drain(0)
            compute_store(2 * g, 0)

            @pl.when(g < n_pairs - 1)
            def _():
                load_and_issue(2 * g + 2, 0)

            drain(1)
            compute_store(2 * g + 1, 1)
            return carry

        lax.fori_loop(0, n_pairs, pair_body, 0)

    return sc_embed


_sc_embed_204800 = _make_sc_embed(1024 * SEQ)


def kernel(x, time, token_table, time_table):
    b, s = x.shape
    xf = x.reshape(-1)
    tf = time.reshape(-1)
    tt8 = (time_table * jnp.float32(SCALE)).astype(jnp.float32)
    pe8 = _pe_scaled_ext()
    out = _sc_embed_204800(xf, tf, token_table, tt8, pe8)
    return out.reshape(b, s, D_MODEL)


# speedup vs baseline: 1.0101x; 1.0101x over previous
import functools
import math

import jax
import jax.numpy as jnp
import numpy as np
from jax import lax
from jax.experimental import pallas as pl
from jax.experimental.pallas import tpu as pltpu
from jax.experimental.pallas import tpu_sc as plsc

D_MODEL = 64
SEQ = 200
NT = 49
SCALE = 8.0
NC = 2
NS = 16
NW = NC * NS
CH = 128
LANES = 16


def _pe_scaled():
    position = np.arange(0, SEQ, dtype=np.float32)[:, None]
    div = np.exp(
        np.arange(0, D_MODEL, 2, dtype=np.float32) * -(math.log(10000.0) / D_MODEL)
    )
    pe = np.zeros((SEQ, D_MODEL), dtype=np.float32)
    pe[:, 0::2] = np.sin(position * div)
    pe[:, 1::2] = np.cos(position * div)
    return jnp.asarray(pe * np.float32(SCALE))


def _make_sc_embed(n_rows):
    rows_per_w = n_rows // NW
    n_chunks = rows_per_w // CH
    n_pairs = n_chunks // 2
    mesh = plsc.VectorSubcoreMesh(core_axis_name="c", subcore_axis_name="s")

    @functools.partial(
        pl.kernel,
        out_type=jax.ShapeDtypeStruct((n_rows // 2, 2 * D_MODEL), jnp.float32),
        mesh=mesh,
        compiler_params=pltpu.CompilerParams(use_tc_tiling_on_sc=True),
        scratch_types=[
            pltpu.VMEM((2, CH), jnp.int32),
            pltpu.VMEM((2, CH), jnp.int32),
            pltpu.VMEM((2, CH), jnp.int32),
            pltpu.VMEM((2, CH, D_MODEL), jnp.float32),
            pltpu.VMEM((2, CH, D_MODEL), jnp.float32),
            pltpu.VMEM((CH // 2, 2 * D_MODEL), jnp.float32),
            pltpu.SemaphoreType.DMA,
            pltpu.SemaphoreType.DMA,
            pltpu.SemaphoreType.DMA,
            pltpu.SemaphoreType.DMA,
        ],
    )
    def sc_embed(xf, tf, tok_tab, comb, out,
                 xi_v, t_v, ci_v, tok_v, comb_v, out_v,
                 sem_t0, sem_t1, sem_m0, sem_m1):
        wid = lax.axis_index("s") * NC + lax.axis_index("c")
        base0 = wid * rows_per_w
        lane = lax.iota(jnp.int32, LANES)
        sems_t = (sem_t0, sem_t1)
        sems_m = (sem_m0, sem_m1)

        def load_and_issue(c, p):
            base = pl.multiple_of(base0 + c * CH, CH)
            s_off = lax.rem(base, SEQ)
            pltpu.sync_copy(xf.at[pl.ds(base, CH)], xi_v.at[p])
            pltpu.sync_copy(tf.at[pl.ds(base, CH)], t_v.at[p])
            for k in range(CH // LANES):
                sl = pl.ds(k * LANES, LANES)
                ci_v[p, sl] = lax.rem(s_off + k * LANES + lane, SEQ) * NT + t_v[p, sl]
            for k in range(CH // LANES):
                vx = xi_v[p, pl.ds(k * LANES, LANES)]
                vc = ci_v[p, pl.ds(k * LANES, LANES)]
                for u in range(LANES):
                    r = k * LANES + u
                    pltpu.async_copy(tok_tab.at[vx[u]], tok_v.at[p, r], sems_t[p])
                    pltpu.async_copy(comb.at[vc[u]], comb_v.at[p, r], sems_m[p])

        def drain(p):
            pltpu.make_async_copy(
                tok_tab.at[pl.ds(0, CH)], tok_v.at[p], sems_t[p]
            ).wait()
            pltpu.make_async_copy(
                comb.at[pl.ds(0, CH)], comb_v.at[p], sems_m[p]
            ).wait()

        def compute_store(c, p):
            base = pl.multiple_of(base0 + c * CH, CH)

            def row_body(r, rcarry):
                ohalf = (r & 1) * D_MODEL
                for j in range(D_MODEL // LANES):
                    sl = pl.ds(j * LANES, LANES)
                    out_v[r >> 1, pl.ds(ohalf + j * LANES, LANES)] = (
                        tok_v[p, r, sl] * SCALE + comb_v[p, r, sl]
                    )
                return rcarry

            lax.fori_loop(0, CH, row_body, 0)
            pltpu.sync_copy(
                out_v, out.at[pl.ds(pl.multiple_of(base // 2, CH // 2), CH // 2)]
            )

        load_and_issue(0, 0)

        def pair_body(g, carry):
            load_and_issue(2 * g + 1, 1)
            drain(0)
            compute_store(2 * g, 0)

            @pl.when(g < n_pairs - 1)
            def _():
                load_and_issue(2 * g + 2, 0)

            drain(1)
            compute_store(2 * g + 1, 1)
            return carry

        lax.fori_loop(0, n_pairs, pair_body, 0)

    return sc_embed


_sc_embed_204800 = _make_sc_embed(1024 * SEQ)


def kernel(x, time, token_table, time_table):
    b, s = x.shape
    xf = x.reshape(-1)
    tf = time.reshape(-1)
    pe8 = _pe_scaled()
    comb = (pe8[:, None, :] + time_table[None, :, :] * jnp.float32(SCALE)).reshape(
        SEQ * NT, D_MODEL
    )
    out = _sc_embed_204800(xf, tf, token_table, comb)
    return out.reshape(b, s, D_MODEL)


# speedup vs baseline: 1.1423x; 1.1308x over previous
import functools
import math

import jax
import jax.numpy as jnp
import numpy as np
from jax import lax
from jax.experimental import pallas as pl
from jax.experimental.pallas import tpu as pltpu
from jax.experimental.pallas import tpu_sc as plsc

D_MODEL = 64
SEQ = 200
NT = 49
SCALE = 8.0
NC = 2
NS = 16
NW = NC * NS
CH = 128
LANES = 16


def _pe_scaled():
    position = np.arange(0, SEQ, dtype=np.float32)[:, None]
    div = np.exp(
        np.arange(0, D_MODEL, 2, dtype=np.float32) * -(math.log(10000.0) / D_MODEL)
    )
    pe = np.zeros((SEQ, D_MODEL), dtype=np.float32)
    pe[:, 0::2] = np.sin(position * div)
    pe[:, 1::2] = np.cos(position * div)
    return jnp.asarray(pe * np.float32(SCALE))


def _make_sc_embed(n_rows):
    rows_per_w = n_rows // NW
    n_chunks = rows_per_w // CH
    n_pairs = n_chunks // 2
    mesh = plsc.VectorSubcoreMesh(core_axis_name="c", subcore_axis_name="s")

    @functools.partial(
        pl.kernel,
        out_type=jax.ShapeDtypeStruct((n_rows // 2, 2 * D_MODEL), jnp.float32),
        mesh=mesh,
        compiler_params=pltpu.CompilerParams(use_tc_tiling_on_sc=True),
        scratch_types=[
            pltpu.VMEM((2, CH), jnp.int32),
            pltpu.VMEM((2, CH), jnp.int32),
            pltpu.VMEM((2, CH), jnp.int32),
            pltpu.VMEM((2, CH, D_MODEL), jnp.float32),
            pltpu.VMEM((2, CH, 2 * D_MODEL), jnp.float32),
            pltpu.VMEM((2, CH // 2, 2 * D_MODEL), jnp.float32),
            pltpu.SemaphoreType.DMA,
            pltpu.SemaphoreType.DMA,
            pltpu.SemaphoreType.DMA,
            pltpu.SemaphoreType.DMA,
            pltpu.SemaphoreType.DMA,
            pltpu.SemaphoreType.DMA,
            pltpu.SemaphoreType.DMA,
            pltpu.SemaphoreType.DMA,
        ],
    )
    def sc_embed(xf, tf, tok_tab, comb, out,
                 xi_v, t_v, ci_v, tok_v, comb_v, out_v,
                 sem_t0, sem_t1, sem_m0, sem_m1,
                 sem_i0, sem_i1, sem_s0, sem_s1):
        wid = lax.axis_index("s") * NC + lax.axis_index("c")
        base0 = wid * rows_per_w
        lane = lax.iota(jnp.int32, LANES)
        sems_t = (sem_t0, sem_t1)
        sems_m = (sem_m0, sem_m1)
        sems_i = (sem_i0, sem_i1)
        sems_s = (sem_s0, sem_s1)

        def chunk_base(c):
            return pl.multiple_of(base0 + c * CH, CH)

        def prefetch_idx(c, p):
            base = chunk_base(c)
            pltpu.async_copy(xf.at[pl.ds(base, CH)], xi_v.at[p], sems_i[p])
            pltpu.async_copy(tf.at[pl.ds(base, CH)], t_v.at[p], sems_i[p])

        def wait_idx(p):
            pltpu.make_async_copy(xf.at[pl.ds(0, CH)], xi_v.at[p], sems_i[p]).wait()
            pltpu.make_async_copy(tf.at[pl.ds(0, CH)], t_v.at[p], sems_i[p]).wait()

        def issue(c, p):
            s_off = lax.rem(chunk_base(c), SEQ)
            for k in range(CH // LANES):
                sl = pl.ds(k * LANES, LANES)
                ci_v[p, sl] = lax.rem(s_off + k * LANES + lane, SEQ) * NT + t_v[p, sl]
            pltpu.async_copy(comb.at[ci_v.at[p]], comb_v.at[p], sems_m[p])
            for k in range(CH // LANES):
                v = xi_v[p, pl.ds(k * LANES, LANES)]
                for u in range(LANES):
                    pltpu.async_copy(
                        tok_tab.at[v[u]], tok_v.at[p, k * LANES + u], sems_t[p]
                    )

        def drain(p):
            pltpu.make_async_copy(
                tok_tab.at[pl.ds(0, CH)], tok_v.at[p], sems_t[p]
            ).wait()
            pltpu.make_async_copy(
                comb.at[pl.ds(0, CH)], comb_v.at[p], sems_m[p]
            ).wait()

        def drain_store(p):
            pltpu.make_async_copy(
                out.at[pl.ds(0, CH // 2)], out_v.at[p], sems_s[p]
            ).wait()

        def compute_store(c, p):
            base = chunk_base(c)

            @pl.when(c >= 2)
            def _():
                drain_store(p)

            def row_body(r, rcarry):
                ohalf = (r & 1) * D_MODEL
                for j in range(D_MODEL // LANES):
                    sl = pl.ds(j * LANES, LANES)
                    out_v[p, r >> 1, pl.ds(ohalf + j * LANES, LANES)] = (
                        tok_v[p, r, sl] * SCALE + comb_v[p, r, sl]
                    )
                return rcarry

            lax.fori_loop(0, CH, row_body, 0)
            pltpu.async_copy(
                out_v.at[p],
                out.at[pl.ds(pl.multiple_of(base // 2, CH // 2), CH // 2)],
                sems_s[p],
            )

        prefetch_idx(0, 0)
        prefetch_idx(1, 1)
        wait_idx(0)
        issue(0, 0)

        def pair_body(g, carry):
            wait_idx(1)
            issue(2 * g + 1, 1)

            @pl.when(g < n_pairs - 1)
            def _():
                prefetch_idx(2 * g + 2, 0)

            drain(0)
            compute_store(2 * g, 0)

            @pl.when(g < n_pairs - 1)
            def _():
                wait_idx(0)
                issue(2 * g + 2, 0)
                prefetch_idx(2 * g + 3, 1)

            drain(1)
            compute_store(2 * g + 1, 1)
            return carry

        lax.fori_loop(0, n_pairs, pair_body, 0)
        drain_store(0)
        drain_store(1)

    return sc_embed


_sc_embed_204800 = _make_sc_embed(1024 * SEQ)


def kernel(x, time, token_table, time_table):
    b, s = x.shape
    xf = x.reshape(-1)
    tf = time.reshape(-1)
    pe8 = _pe_scaled()
    comb = pe8[:, None, :] + time_table[None, :, :] * jnp.float32(SCALE)
    comb = jnp.pad(comb.reshape(SEQ * NT, D_MODEL), ((0, 0), (0, D_MODEL)))
    out = _sc_embed_204800(xf, tf, token_table, comb)
    return out.reshape(b, s, D_MODEL)


# speedup vs baseline: 1.4383x; 1.2591x over previous
import functools
import math

import jax
import jax.numpy as jnp
import numpy as np
from jax import lax
from jax.experimental import pallas as pl
from jax.experimental.pallas import tpu as pltpu
from jax.experimental.pallas import tpu_sc as plsc

D_MODEL = 64
SEQ = 200
NT = 49
SCALE = 8.0
NC = 2
NS = 16
NW = NC * NS
CH = 128
LANES = 16


def _pe_scaled():
    position = np.arange(0, SEQ, dtype=np.float32)[:, None]
    div = np.exp(
        np.arange(0, D_MODEL, 2, dtype=np.float32) * -(math.log(10000.0) / D_MODEL)
    )
    pe = np.zeros((SEQ, D_MODEL), dtype=np.float32)
    pe[:, 0::2] = np.sin(position * div)
    pe[:, 1::2] = np.cos(position * div)
    return jnp.asarray(pe * np.float32(SCALE))


def _make_sc_embed(n_rows):
    rows_per_w = n_rows // NW
    n_chunks = rows_per_w // CH
    n_pairs = n_chunks // 2
    mesh = plsc.VectorSubcoreMesh(core_axis_name="c", subcore_axis_name="s")

    @functools.partial(
        pl.kernel,
        out_type=jax.ShapeDtypeStruct((n_rows, D_MODEL), jnp.float32),
        mesh=mesh,
        compiler_params=pltpu.CompilerParams(use_tc_tiling_on_sc=True),
        scratch_types=[
            pltpu.VMEM((2, CH), jnp.int32),
            pltpu.VMEM((2, CH), jnp.int32),
            pltpu.VMEM((2, CH), jnp.int32),
            pltpu.VMEM((2, CH, D_MODEL), jnp.float32),
            pltpu.VMEM((2, CH, 2 * D_MODEL), jnp.float32),
            pltpu.VMEM((2, CH, D_MODEL), jnp.float32),
            pltpu.SemaphoreType.DMA,
            pltpu.SemaphoreType.DMA,
            pltpu.SemaphoreType.DMA,
            pltpu.SemaphoreType.DMA,
            pltpu.SemaphoreType.DMA,
            pltpu.SemaphoreType.DMA,
            pltpu.SemaphoreType.DMA,
            pltpu.SemaphoreType.DMA,
        ],
    )
    def sc_embed(xf, tf, tok_tab, comb, out,
                 xi_v, t_v, ci_v, tok_v, comb_v, out_v,
                 sem_t0, sem_t1, sem_m0, sem_m1,
                 sem_i0, sem_i1, sem_s0, sem_s1):
        wid = lax.axis_index("s") * NC + lax.axis_index("c")
        base0 = wid * rows_per_w
        lane = lax.iota(jnp.int32, LANES)
        sems_t = (sem_t0, sem_t1)
        sems_m = (sem_m0, sem_m1)
        sems_i = (sem_i0, sem_i1)
        sems_s = (sem_s0, sem_s1)

        def chunk_base(c):
            return pl.multiple_of(base0 + c * CH, CH)

        def prefetch_idx(c, p):
            base = chunk_base(c)
            pltpu.async_copy(xf.at[pl.ds(base, CH)], xi_v.at[p], sems_i[p])
            pltpu.async_copy(tf.at[pl.ds(base, CH)], t_v.at[p], sems_i[p])

        def wait_idx(p):
            pltpu.make_async_copy(xf.at[pl.ds(0, CH)], xi_v.at[p], sems_i[p]).wait()
            pltpu.make_async_copy(tf.at[pl.ds(0, CH)], t_v.at[p], sems_i[p]).wait()

        def issue(c, p):
            s_off = lax.rem(chunk_base(c), SEQ)
            for k in range(CH // LANES):
                sl = pl.ds(k * LANES, LANES)
                ci_v[p, sl] = lax.rem(s_off + k * LANES + lane, SEQ) * NT + t_v[p, sl]
            pltpu.async_copy(comb.at[ci_v.at[p]], comb_v.at[p], sems_m[p])
            for k in range(CH // LANES):
                v = xi_v[p, pl.ds(k * LANES, LANES)]
                for u in range(LANES):
                    pltpu.async_copy(
                        tok_tab.at[v[u]], tok_v.at[p, k * LANES + u], sems_t[p]
                    )

        def drain(p):
            pltpu.make_async_copy(
                tok_tab.at[pl.ds(0, CH)], tok_v.at[p], sems_t[p]
            ).wait()
            pltpu.make_async_copy(
                comb.at[pl.ds(0, CH)], comb_v.at[p], sems_m[p]
            ).wait()

        def drain_store(p):
            pltpu.make_async_copy(
                out.at[pl.ds(0, CH)], out_v.at[p], sems_s[p]
            ).wait()

        def compute_store(c, p):
            base = chunk_base(c)

            @pl.when(c >= 2)
            def _():
                drain_store(p)

            def row_body(r, rcarry):
                for j in range(D_MODEL // LANES):
                    sl = pl.ds(j * LANES, LANES)
                    out_v[p, r, sl] = tok_v[p, r, sl] * SCALE + comb_v[p, r, sl]
                return rcarry

            lax.fori_loop(0, CH, row_body, 0)
            pltpu.async_copy(out_v.at[p], out.at[pl.ds(base, CH)], sems_s[p])

        prefetch_idx(0, 0)
        prefetch_idx(1, 1)
        wait_idx(0)
        issue(0, 0)

        def pair_body(g, carry):
            wait_idx(1)
            issue(2 * g + 1, 1)

            @pl.when(g < n_pairs - 1)
            def _():
                prefetch_idx(2 * g + 2, 0)

            drain(0)
            compute_store(2 * g, 0)

            @pl.when(g < n_pairs - 1)
            def _():
                wait_idx(0)
                issue(2 * g + 2, 0)
                prefetch_idx(2 * g + 3, 1)

            drain(1)
            compute_store(2 * g + 1, 1)
            return carry

        lax.fori_loop(0, n_pairs, pair_body, 0)
        drain_store(0)
        drain_store(1)

    return sc_embed


_sc_embed_204800 = _make_sc_embed(1024 * SEQ)


def kernel(x, time, token_table, time_table):
    b, s = x.shape
    xf = x.reshape(-1)
    tf = time.reshape(-1)
    pe8 = _pe_scaled()
    comb = pe8[:, None, :] + time_table[None, :, :] * jnp.float32(SCALE)
    comb = jnp.pad(comb.reshape(SEQ * NT, D_MODEL), ((0, 0), (0, D_MODEL)))
    out = _sc_embed_204800(xf, tf, token_table, comb)
    return out.reshape(b, s, D_MODEL)
